# Initial kernel scaffold; baseline (speedup 1.0000x reference)
import functools

import jax
import jax.numpy as jnp
from jax import lax
from jax.experimental import pallas as pl
from jax.experimental.pallas import tpu as pltpu
from jax.experimental.pallas import tpu_sc as plsc

_NC = 2
_NS = 16
_NW = _NC * _NS


def _emb_gather(total, D, n_chunks, chunk):
    b_per_w = total // _NW
    mesh = plsc.VectorSubcoreMesh(core_axis_name="c", subcore_axis_name="s")

    @functools.partial(
        pl.kernel,
        mesh=mesh,
        out_type=jax.ShapeDtypeStruct((total, D), jnp.float32),
        scratch_types=[
            pltpu.VMEM((chunk,), jnp.int32),
            pltpu.VMEM((chunk,), jnp.int32),
            pltpu.VMEM((chunk, D), jnp.float32),
            pltpu.VMEM((chunk, D), jnp.float32),
            pltpu.SemaphoreType.DMA,
            pltpu.SemaphoreType.DMA,
            pltpu.SemaphoreType.DMA,
            pltpu.SemaphoreType.DMA,
        ],
    )
    def body(table, idx, out, idx0, idx1, rows0, rows1, g0, g1, w0, w1):
        wid = lax.axis_index("s") * _NC + lax.axis_index("c")
        base = wid * b_per_w
        idx_bufs = (idx0, idx1)
        row_bufs = (rows0, rows1)
        gsems = (g0, g1)
        wsems = (w0, w1)

        gathers = [None] * n_chunks
        writes = [None] * n_chunks

        pltpu.sync_copy(idx.at[pl.ds(base, chunk)], idx_bufs[0])
        gathers[0] = pltpu.async_copy(table.at[idx_bufs[0]], row_bufs[0], gsems[0])
        for c in range(1, n_chunks + 1):
            s = c & 1
            if c < n_chunks:
                pltpu.sync_copy(
                    idx.at[pl.ds(base + c * chunk, chunk)], idx_bufs[s])
                if c >= 2:
                    writes[c - 2].wait()
                gathers[c] = pltpu.async_copy(
                    table.at[idx_bufs[s]], row_bufs[s], gsems[s])
            p = (c - 1) & 1
            gathers[c - 1].wait()
            writes[c - 1] = pltpu.async_copy(
                row_bufs[p], out.at[pl.ds(base + (c - 1) * chunk, chunk)],
                wsems[p])
        writes[n_chunks - 2].wait()
        writes[n_chunks - 1].wait()

    return body


def kernel(weights, indices):
    B, F = indices.shape
    V, D = weights.shape
    total = B * F
    idx_flat = indices.reshape(total).astype(jnp.int32)

    b_per_w = total // _NW
    n_chunks = 8
    chunk = b_per_w // n_chunks

    out = _emb_gather(total, D, n_chunks, chunk)(weights, idx_flat)
    return out.reshape(B, F, D)


# speedup vs baseline: 1.3541x; 1.3541x over previous
import functools

import jax
import jax.numpy as jnp
from jax import lax
from jax.experimental import pallas as pl
from jax.experimental.pallas import tpu as pltpu
from jax.experimental.pallas import tpu_sc as plsc

_NC = 2
_NS = 16
_NW = _NC * _NS


def _emb_gather(V, D, N, F, chn):
    n_per_w = N // _NW
    nch = n_per_w // chn
    mesh = plsc.VectorSubcoreMesh(core_axis_name="c", subcore_axis_name="s")

    @functools.partial(
        pl.kernel,
        mesh=mesh,
        out_type=jax.ShapeDtypeStruct((N, F, D), jnp.float32),
        compiler_params=pltpu.CompilerParams(use_tc_tiling_on_sc=False),
        scratch_types=[
            pltpu.VMEM((chn, F), jnp.int32),
            pltpu.VMEM((chn, F), jnp.int32),
            pltpu.VMEM((chn, F, D), jnp.float32),
            pltpu.VMEM((chn, F, D), jnp.float32),
            pltpu.SemaphoreType.DMA,
            pltpu.SemaphoreType.DMA,
            pltpu.SemaphoreType.DMA,
            pltpu.SemaphoreType.DMA,
        ],
    )
    def body(table, idx2, out, i0, i1, r0, r1, g0, g1, w0, w1):
        wid = lax.axis_index("s") * _NC + lax.axis_index("c")
        base = wid * n_per_w
        idx_bufs = (i0, i1)
        row_bufs = (r0, r1)
        gsems = (g0, g1)
        wsems = (w0, w1)

        def fire(s):
            def one(a, _):
                pltpu.async_copy(
                    table.at[idx_bufs[s].at[a]], row_bufs[s].at[a], gsems[s])
                return 0
            lax.fori_loop(0, chn, one, 0)

        def drain(s):
            pltpu.make_async_copy(
                out.at[pl.ds(0, chn)], row_bufs[s], gsems[s]).wait()

        writes = [None] * nch
        pltpu.sync_copy(idx2.at[pl.ds(base, chn)], idx_bufs[0])
        fire(0)
        for c in range(1, nch + 1):
            s = c & 1
            if c < nch:
                pltpu.sync_copy(
                    idx2.at[pl.ds(base + c * chn, chn)], idx_bufs[s])
                if c >= 2:
                    writes[c - 2].wait()
                fire(s)
            p = (c - 1) & 1
            drain(p)
            writes[c - 1] = pltpu.async_copy(
                row_bufs[p], out.at[pl.ds(base + (c - 1) * chn, chn)],
                wsems[p])
        writes[nch - 2].wait()
        writes[nch - 1].wait()

    return body


def kernel(weights, indices):
    N, F = indices.shape
    V, D = weights.shape
    idx = indices.astype(jnp.int32)
    return _emb_gather(V, D, N, F, 64)(weights, idx)
